# Initial kernel scaffold; baseline (speedup 1.0000x reference)
import jax
import jax.numpy as jnp
from jax.experimental import pallas as pl


def kernel(delta_m, delta_embed_weight):
    raise NotImplementedError("write your pallas kernel here")


# speedup vs baseline: 2.0232x; 2.0232x over previous
import functools

import jax
import jax.numpy as jnp
from jax import lax
from jax.experimental import pallas as pl
from jax.experimental.pallas import tpu as pltpu
from jax.experimental.pallas import tpu_sc as plsc

MAXD = 10
EDIM = 32
ROWS = 2 * MAXD + 1

L = 16
NC = 2
NS = 16
NW = NC * NS

CHUNK = 1024


def _sc_lookup(idx_flat, tab_flat, B):
    b_per_w = B // NW
    n_chunks = b_per_w // CHUNK
    mesh = plsc.VectorSubcoreMesh(core_axis_name="c", subcore_axis_name="s")

    @functools.partial(
        pl.kernel,
        mesh=mesh,
        compiler_params=pltpu.CompilerParams(needs_layout_passes=False),
        out_type=jax.ShapeDtypeStruct((B * EDIM,), jnp.float32),
        scratch_types=[
            pltpu.VMEM((ROWS * EDIM,), jnp.float32),
            pltpu.VMEM((CHUNK,), jnp.int32),
            pltpu.VMEM((CHUNK * EDIM,), jnp.float32),
        ],
    )
    def k(idx_hbm, tab_hbm, out_hbm, tab_v, idx_v, rows_v):
        wid = lax.axis_index("s") * NC + lax.axis_index("c")
        base = wid * b_per_w
        pltpu.sync_copy(tab_hbm, tab_v)
        lane = lax.iota(jnp.int32, L)
        lane32 = lane * EDIM

        def chunk_body(c, _):
            off = base + c * CHUNK
            pltpu.sync_copy(idx_hbm.at[pl.ds(off, CHUNK)], idx_v)

            def group_body(g, _):
                raw = idx_v[pl.ds(g * L, L)]
                t = jnp.minimum(jnp.maximum(raw + MAXD, 0), ROWS - 1) * EDIM
                pos0 = g * (L * EDIM) + lane32
                for d in range(EDIM):
                    v = plsc.load_gather(tab_v, [t + d])
                    plsc.store_scatter(rows_v, [pos0 + d], v)
                return 0

            lax.fori_loop(0, CHUNK // L, group_body, 0)
            pltpu.sync_copy(rows_v, out_hbm.at[pl.ds(off * EDIM, CHUNK * EDIM)])
            return 0

        lax.fori_loop(0, n_chunks, chunk_body, 0)

    return k(idx_flat, tab_flat)


def kernel(delta_m, delta_embed_weight):
    B = delta_m.shape[0] * delta_m.shape[1]
    idx_flat = delta_m.reshape(-1).astype(jnp.int32)
    tab_flat = delta_embed_weight.reshape(-1).astype(jnp.float32)
    out_flat = _sc_lookup(idx_flat, tab_flat, B)
    return out_flat.reshape(delta_m.shape + (EDIM,))


# speedup vs baseline: 4.8708x; 2.4075x over previous
import functools

import jax
import jax.numpy as jnp
from jax import lax
from jax.experimental import pallas as pl
from jax.experimental.pallas import tpu as pltpu
from jax.experimental.pallas import tpu_sc as plsc

MAXD = 10
EDIM = 32
ROWS = 2 * MAXD + 1

L = 16
NC = 2
NS = 16
NW = NC * NS

CHUNK = 1024
NBUF = 3


def _sc_lookup(idx_flat, tab, B):
    b_per_w = B // NW
    n_chunks = b_per_w // CHUNK
    mesh = plsc.VectorSubcoreMesh(core_axis_name="c", subcore_axis_name="s")

    @functools.partial(
        pl.kernel,
        mesh=mesh,
        compiler_params=pltpu.CompilerParams(
            needs_layout_passes=False, use_tc_tiling_on_sc=False
        ),
        out_type=jax.ShapeDtypeStruct((B, EDIM), jnp.float32),
        scratch_types=[
            pltpu.VMEM((ROWS, EDIM), jnp.float32),
            pltpu.MemorySpace.VMEM_SHARED((ROWS, EDIM), jnp.float32),
            pltpu.VMEM((b_per_w,), jnp.int32),
            [pltpu.VMEM((CHUNK, EDIM), jnp.float32) for _ in range(NBUF)],
            pltpu.SemaphoreType.DMA,
            [pltpu.SemaphoreType.DMA for _ in range(NBUF)],
        ],
    )
    def k(idx_hbm, tab_hbm, out_hbm, tab_v, tab_sh, idx_v, rows, gsem, osems):
        sid = lax.axis_index("s")
        wid = sid * NC + lax.axis_index("c")
        base = wid * b_per_w

        pltpu.sync_copy(tab_hbm, tab_v)

        @pl.when(sid == 0)
        def _():
            pltpu.sync_copy(tab_v, tab_sh)

        plsc.subcore_barrier()

        pltpu.sync_copy(idx_hbm.at[pl.ds(base, b_per_w)], idx_v)

        def tbody(g, _):
            raw = idx_v[pl.ds(g * L, L)]
            idx_v[pl.ds(g * L, L)] = jnp.minimum(
                jnp.maximum(raw + MAXD, 0), ROWS - 1
            )
            return 0

        lax.fori_loop(0, b_per_w // L, tbody, 0)

        out_handles = [None] * n_chunks
        for c in range(n_chunks):
            b = c % NBUF
            if c >= NBUF:
                out_handles[c - NBUF].wait()
            pltpu.async_copy(
                tab_sh.at[idx_v.at[pl.ds(c * CHUNK, CHUNK)]], rows[b], gsem
            ).wait()
            out_handles[c] = pltpu.async_copy(
                rows[b], out_hbm.at[pl.ds(base + c * CHUNK, CHUNK)], osems[b]
            )
        for c in range(n_chunks - NBUF, n_chunks):
            out_handles[c].wait()

    return k(idx_flat, tab)


def kernel(delta_m, delta_embed_weight):
    B = delta_m.shape[0] * delta_m.shape[1]
    idx_flat = delta_m.reshape(-1).astype(jnp.int32)
    tab = delta_embed_weight.astype(jnp.float32)
    out = _sc_lookup(idx_flat, tab, B)
    return out.reshape(delta_m.shape + (EDIM,))
